# baseline (speedup 1.0000x reference)
import functools

import jax
import jax.numpy as jnp
from jax import lax
from jax.experimental import pallas as pl
from jax.experimental.pallas import tpu as pltpu
from jax.experimental.pallas import tpu_sc as plsc

B = 16384
D = 64
NP = 65536
NF = 10000
NC = 2
NS = 16
L = 16
NW = NC * NS
BPW = B // NW
CH = 64
NCH = BPW // CH
PBLK = 2048
FBLK = 1000


def _prep_px_body(ut_ref, p1t_ref, p2t_ref, wa_ref, wc_ref, pk_ref, t_ref):
    wc = wc_ref[...]
    a1 = jnp.maximum(p1t_ref[...], 0.0) * wc
    a2 = jnp.maximum(p2t_ref[...], 0.0) * wc
    eye = (lax.broadcasted_iota(jnp.int32, (D, D), 0)
           == lax.broadcasted_iota(jnp.int32, (D, D), 1)).astype(jnp.float32)
    dn = (((0,), (0,)), ((), ()))
    a1t = lax.dot_general(a1, eye, dn, precision=None)
    a2t = lax.dot_general(a2, eye, dn, precision=None)
    pk_ref[...] = jnp.concatenate([a1t, a2t], axis=1)
    t_ref[...] = jnp.sum(ut_ref[...] * wa_ref[...], axis=0)[None, None, :]


def _prep_px(ut, p1t, p2t, wa, wc):
    g = NP // PBLK
    spec = pl.BlockSpec((D, PBLK), lambda i: (0, i))
    wspec = pl.BlockSpec((D, 1), lambda i: (0, 0))
    pk, t = pl.pallas_call(
        _prep_px_body,
        grid=(g,),
        in_specs=[spec, spec, spec, wspec, wspec],
        out_specs=[pl.BlockSpec((PBLK, 2 * D), lambda i: (i, 0)),
                   pl.BlockSpec((1, 1, PBLK), lambda i: (i, 0, 0))],
        out_shape=[jax.ShapeDtypeStruct((NP, 2 * D), jnp.float32),
                   jax.ShapeDtypeStruct((g, 1, PBLK), jnp.float32)],
    )(ut, p1t, p2t, wa, wc)
    return pk, t.reshape(NP)


def _prep_fr_body(vt_ref, p1t_ref, p2t_ref, wb_ref, pk_ref, t_ref):
    a1 = jnp.maximum(p1t_ref[...], 0.0)
    a2 = jnp.maximum(p2t_ref[...], 0.0)
    eye = (lax.broadcasted_iota(jnp.int32, (D, D), 0)
           == lax.broadcasted_iota(jnp.int32, (D, D), 1)).astype(jnp.float32)
    dn = (((0,), (0,)), ((), ()))
    a1t = lax.dot_general(a1, eye, dn, precision=None)
    a2t = lax.dot_general(a2, eye, dn, precision=None)
    pk_ref[...] = jnp.concatenate([a1t, a2t], axis=1)
    t_ref[...] = jnp.sum(vt_ref[...] * wb_ref[...], axis=0)[None, None, :]


def _prep_fr(vt, p1t, p2t, wb):
    spec = pl.BlockSpec((D, NF), lambda: (0, 0))
    wspec = pl.BlockSpec((D, 1), lambda: (0, 0))
    pk, t = pl.pallas_call(
        _prep_fr_body,
        in_specs=[spec, spec, spec, wspec],
        out_specs=[pl.BlockSpec((NF, 2 * D), lambda: (0, 0)),
                   pl.BlockSpec((1, 1, NF), lambda: (0, 0, 0))],
        out_shape=[jax.ShapeDtypeStruct((NF, 2 * D), jnp.float32),
                   jax.ShapeDtypeStruct((1, 1, NF), jnp.float32)],
    )(vt, p1t, p2t, wb)
    return pk, t.reshape(NF)


def _sc_body(pixel_h, frame_h, wv_h, sw_h, tgt_h, upk_h, vpk_h, tu_h, tv_h,
             x_h, s_h,
             idxp, idxf, bu, bv, btu, btv, btg, wv, sw, zbuf, xbuf, sbuf,
             sems):
    wid = lax.axis_index("s") * NC + lax.axis_index("c")
    base = wid * BPW
    pltpu.sync_copy(wv_h, wv)
    pltpu.sync_copy(sw_h, sw)
    iota = lax.iota(jnp.int32, L)
    m15 = iota == jnp.full((L,), L - 1, jnp.int32)
    zero = jnp.zeros((L,), jnp.float32)
    one = jnp.full((L,), 1.0, jnp.float32)
    w2v = wv[0]
    b2v = wv[1]

    pltpu.sync_copy(pixel_h.at[pl.ds(base, BPW)], idxp)
    pltpu.sync_copy(frame_h.at[pl.ds(base, BPW)], idxf)
    pltpu.sync_copy(tgt_h.at[pl.ds(base, BPW)], btg)

    def fire(c, s):
        ip = idxp.at[pl.ds(c * CH, CH)]
        if_ = idxf.at[pl.ds(c * CH, CH)]
        pltpu.async_copy(upk_h.at[ip], bu.at[s], sems.at[s])
        pltpu.async_copy(vpk_h.at[if_], bv.at[s], sems.at[s])
        pltpu.async_copy(tu_h.at[ip], btu.at[pl.ds(c * CH, CH)], sems.at[s])
        pltpu.async_copy(tv_h.at[if_], btv.at[pl.ds(c * CH, CH)], sems.at[s])

    def drain(s):
        ip = idxp.at[pl.ds(0, CH)]
        if_ = idxf.at[pl.ds(0, CH)]
        pltpu.make_async_copy(upk_h.at[ip], bu.at[s], sems.at[s]).wait()
        pltpu.make_async_copy(vpk_h.at[if_], bv.at[s], sems.at[s]).wait()
        pltpu.make_async_copy(tu_h.at[ip], btu.at[pl.ds(0, CH)],
                              sems.at[s]).wait()
        pltpu.make_async_copy(tv_h.at[if_], btv.at[pl.ds(0, CH)],
                              sems.at[s]).wait()

    fire(0, 0)
    for c in range(NCH):
        s = c % 2
        drain(s)
        if c + 1 < NCH:
            fire(c + 1, 1 - s)
        bus = bu.at[s]
        bvs = bv.at[s]

        @plsc.parallel_loop(0, CH, step=1)
        def _elem(e):
            p = bus[e, pl.ds(0, L)] * bvs[e, pl.ds(0, L)]
            for k in range(1, 2 * D // L):
                p = p + bus[e, pl.ds(k * L, L)] * bvs[e, pl.ds(k * L, L)]
            cs = plsc.cumsum(p)
            idx = jnp.full((L,), c * CH + e, jnp.int32)
            plsc.store_scatter(zbuf, [idx], cs, mask=m15)

    @plsc.parallel_loop(0, BPW, step=L)
    def _tailg(gbase):
        gl = pl.ds(gbase, L)
        z = zbuf[gl] + btu[gl] + btv[gl]
        h = jnp.maximum(z, zero)
        t = h * w2v + b2v
        xg = one / (one + jnp.exp(-t))
        xbuf[gl] = xg
        sv = btg[gl] - xg
        h1 = [jnp.maximum(sv * sw[k] + sw[10 + k], zero) for k in range(10)]
        h2 = [jnp.maximum(
                  sum(h1[j] * sw[20 + j * 10 + k] for j in range(10))
                  + sw[120 + k], zero) for k in range(10)]
        h3 = [jnp.maximum(
                  sum(h2[j] * sw[130 + j * 10 + k] for j in range(10))
                  + sw[230 + k], zero) for k in range(10)]
        o = sum(h3[j] * sw[240 + j] for j in range(10)) + sw[250]
        sbuf[gl] = one / (one + jnp.exp(-o))

    pltpu.sync_copy(xbuf, x_h.at[pl.ds(base, BPW)])
    pltpu.sync_copy(sbuf, s_h.at[pl.ds(base, BPW)])


_sc_call = functools.partial(
    pl.kernel,
    out_type=[jax.ShapeDtypeStruct((B,), jnp.float32),
              jax.ShapeDtypeStruct((B,), jnp.float32)],
    mesh=plsc.VectorSubcoreMesh(
        core_axis_name="c", subcore_axis_name="s", num_cores=NC,
        num_subcores=NS),
    scratch_types=[
        pltpu.VMEM((BPW,), jnp.int32),
        pltpu.VMEM((BPW,), jnp.int32),
        pltpu.VMEM((2, CH, 2 * D), jnp.float32),
        pltpu.VMEM((2, CH, 2 * D), jnp.float32),
        pltpu.VMEM((BPW,), jnp.float32),
        pltpu.VMEM((BPW,), jnp.float32),
        pltpu.VMEM((BPW,), jnp.float32),
        pltpu.VMEM((8, L), jnp.float32),
        pltpu.VMEM((256, L), jnp.float32),
        pltpu.VMEM((BPW,), jnp.float32),
        pltpu.VMEM((BPW,), jnp.float32),
        pltpu.VMEM((BPW,), jnp.float32),
        pltpu.SemaphoreType.DMA((2,)),
    ],
    compiler_params=pltpu.CompilerParams(
        needs_layout_passes=False, use_tc_tiling_on_sc=False),
)(_sc_body)


def kernel(pixel, frame, target, U, V, Up1, Up2, Vp1, Vp2, W1, b1, W2, b2,
           S1, bs1, S2, bs2, S3, bs3, S4, bs4):
    pixel_i = pixel.astype(jnp.int32)
    frame_i = frame.astype(jnp.int32)
    w1 = W1.reshape(-1)
    wa = w1[:D].reshape(D, 1)
    wb = w1[D:2 * D].reshape(D, 1)
    wc = w1[2 * D:].reshape(D, 1)
    upk, tu = _prep_px(U.T, Up1.T, Up2.T, wa, wc)
    vpk, tv = _prep_fr(V.T, Vp1.T, Vp2.T, wb)
    tv = tv + b1[0]
    wsm = jnp.broadcast_to(
        jnp.concatenate([W2.reshape(-1), b2.reshape(-1),
                         jnp.zeros((6,), jnp.float32)])[:, None], (8, L))
    swf = jnp.concatenate([
        S1.reshape(-1), bs1, S2.reshape(-1), bs2, S3.reshape(-1), bs3,
        S4.reshape(-1), bs4, jnp.zeros((5,), jnp.float32)])
    sw = jnp.broadcast_to(swf[:, None], (256, L))
    x, sx = _sc_call(pixel_i, frame_i, wsm, sw, target.reshape(-1),
                     upk, vpk, tu, tv)
    return (x.reshape(B, 1), sx.reshape(B, 1))


# speedup vs baseline: 1.2238x; 1.2238x over previous
import functools

import jax
import jax.numpy as jnp
from jax import lax
from jax.experimental import pallas as pl
from jax.experimental.pallas import tpu as pltpu
from jax.experimental.pallas import tpu_sc as plsc

B = 16384
D = 64
NP = 65536
NF = 10000
NC = 2
NS = 16
L = 16
NW = NC * NS
BPW = B // NW
CH = 64
NCH = BPW // CH
PBLK = 4096
FBLK = 1000


def _prep_px_body(ut_ref, p1t_ref, p2t_ref, wa_ref, wc_ref, pk_ref, t_ref):
    wc = wc_ref[...]
    a1 = jnp.maximum(p1t_ref[...], 0.0) * wc
    a2 = jnp.maximum(p2t_ref[...], 0.0) * wc
    eye = (lax.broadcasted_iota(jnp.int32, (D, D), 0)
           == lax.broadcasted_iota(jnp.int32, (D, D), 1)).astype(jnp.float32)
    dn = (((0,), (0,)), ((), ()))
    a1t = lax.dot_general(a1, eye, dn, precision=None)
    a2t = lax.dot_general(a2, eye, dn, precision=None)
    pk_ref[...] = jnp.concatenate([a1t, a2t], axis=1)
    t_ref[...] = jnp.sum(ut_ref[...] * wa_ref[...], axis=0)[None, None, :]


def _prep_px(ut, p1t, p2t, wa, wc):
    g = NP // PBLK
    spec = pl.BlockSpec((D, PBLK), lambda i: (0, i))
    wspec = pl.BlockSpec((D, 1), lambda i: (0, 0))
    pk, t = pl.pallas_call(
        _prep_px_body,
        grid=(g,),
        in_specs=[spec, spec, spec, wspec, wspec],
        out_specs=[pl.BlockSpec((PBLK, 2 * D), lambda i: (i, 0)),
                   pl.BlockSpec((1, 1, PBLK), lambda i: (i, 0, 0))],
        out_shape=[jax.ShapeDtypeStruct((NP, 2 * D), jnp.float32),
                   jax.ShapeDtypeStruct((g, 1, PBLK), jnp.float32)],
    )(ut, p1t, p2t, wa, wc)
    return pk, t.reshape(NP)


def _prep_fr_body(vt_ref, p1t_ref, p2t_ref, wb_ref, pk_ref, t_ref):
    a1 = jnp.maximum(p1t_ref[...], 0.0)
    a2 = jnp.maximum(p2t_ref[...], 0.0)
    eye = (lax.broadcasted_iota(jnp.int32, (D, D), 0)
           == lax.broadcasted_iota(jnp.int32, (D, D), 1)).astype(jnp.float32)
    dn = (((0,), (0,)), ((), ()))
    a1t = lax.dot_general(a1, eye, dn, precision=None)
    a2t = lax.dot_general(a2, eye, dn, precision=None)
    pk_ref[...] = jnp.concatenate([a1t, a2t], axis=1)
    t_ref[...] = jnp.sum(vt_ref[...] * wb_ref[...], axis=0)[None, None, :]


def _prep_fr(vt, p1t, p2t, wb):
    spec = pl.BlockSpec((D, NF), lambda: (0, 0))
    wspec = pl.BlockSpec((D, 1), lambda: (0, 0))
    pk, t = pl.pallas_call(
        _prep_fr_body,
        in_specs=[spec, spec, spec, wspec],
        out_specs=[pl.BlockSpec((NF, 2 * D), lambda: (0, 0)),
                   pl.BlockSpec((1, 1, NF), lambda: (0, 0, 0))],
        out_shape=[jax.ShapeDtypeStruct((NF, 2 * D), jnp.float32),
                   jax.ShapeDtypeStruct((1, 1, NF), jnp.float32)],
    )(vt, p1t, p2t, wb)
    return pk, t.reshape(NF)


def _sc_body(pixel_h, frame_h, wv_h, upk_h, vpk_h, tu_h, tv_h,
             x_h,
             idxp, idxf, bu, bv, btu, btv, wv, zbuf, xbuf, sems):
    wid = lax.axis_index("s") * NC + lax.axis_index("c")
    base = wid * BPW
    pltpu.sync_copy(wv_h, wv)
    iota = lax.iota(jnp.int32, L)
    m15 = iota == jnp.full((L,), L - 1, jnp.int32)
    zero = jnp.zeros((L,), jnp.float32)
    one = jnp.full((L,), 1.0, jnp.float32)
    w2v = wv[0]
    b2v = wv[1]

    pltpu.sync_copy(pixel_h.at[pl.ds(base, BPW)], idxp)
    pltpu.sync_copy(frame_h.at[pl.ds(base, BPW)], idxf)

    def fire(c, s):
        ip = idxp.at[pl.ds(c * CH, CH)]
        if_ = idxf.at[pl.ds(c * CH, CH)]
        pltpu.async_copy(upk_h.at[ip], bu.at[s], sems.at[s])
        pltpu.async_copy(vpk_h.at[if_], bv.at[s], sems.at[s])
        pltpu.async_copy(tu_h.at[ip], btu.at[s], sems.at[s])
        pltpu.async_copy(tv_h.at[if_], btv.at[s], sems.at[s])

    def drain(s):
        ip = idxp.at[pl.ds(0, CH)]
        if_ = idxf.at[pl.ds(0, CH)]
        pltpu.make_async_copy(upk_h.at[ip], bu.at[s], sems.at[s]).wait()
        pltpu.make_async_copy(vpk_h.at[if_], bv.at[s], sems.at[s]).wait()
        pltpu.make_async_copy(tu_h.at[ip], btu.at[s], sems.at[s]).wait()
        pltpu.make_async_copy(tv_h.at[if_], btv.at[s], sems.at[s]).wait()

    fire(0, 0)
    for c in range(NCH):
        s = c % 2
        drain(s)
        if c + 1 < NCH:
            fire(c + 1, 1 - s)
        bus = bu.at[s]
        bvs = bv.at[s]

        @plsc.parallel_loop(0, CH, step=1)
        def _elem(e):
            p = bus[e, pl.ds(0, L)] * bvs[e, pl.ds(0, L)]
            for k in range(1, 2 * D // L):
                p = p + bus[e, pl.ds(k * L, L)] * bvs[e, pl.ds(k * L, L)]
            cs = plsc.cumsum(p)
            idx = jnp.full((L,), e, jnp.int32)
            plsc.store_scatter(zbuf, [idx], cs, mask=m15)

        @plsc.parallel_loop(0, CH, step=L)
        def _tailg(gbase):
            gl = pl.ds(gbase, L)
            z = zbuf[gl] + btu[s, gl] + btv[s, gl]
            h = jnp.maximum(z, zero)
            t = h * w2v + b2v
            xbuf[pl.ds(c * CH + gbase, L)] = one / (one + jnp.exp(-t))

    pltpu.sync_copy(xbuf, x_h.at[pl.ds(base, BPW)])


_sc_call = functools.partial(
    pl.kernel,
    out_type=jax.ShapeDtypeStruct((B,), jnp.float32),
    mesh=plsc.VectorSubcoreMesh(
        core_axis_name="c", subcore_axis_name="s", num_cores=NC,
        num_subcores=NS),
    scratch_types=[
        pltpu.VMEM((BPW,), jnp.int32),
        pltpu.VMEM((BPW,), jnp.int32),
        pltpu.VMEM((2, CH, 2 * D), jnp.float32),
        pltpu.VMEM((2, CH, 2 * D), jnp.float32),
        pltpu.VMEM((2, CH), jnp.float32),
        pltpu.VMEM((2, CH), jnp.float32),
        pltpu.VMEM((8, L), jnp.float32),
        pltpu.VMEM((CH,), jnp.float32),
        pltpu.VMEM((BPW,), jnp.float32),
        pltpu.SemaphoreType.DMA((2,)),
    ],
    compiler_params=pltpu.CompilerParams(
        needs_layout_passes=False, use_tc_tiling_on_sc=False),
)(_sc_body)


def _tail_body(x_ref, t_ref, s1, bs1, s2, bs2, s3, bs3, s4, bs4, o_ref):
    s = t_ref[...] - x_ref[...]
    h1 = [jnp.maximum(s * s1[0, k] + bs1[k], 0.0) for k in range(10)]
    h2 = [jnp.maximum(sum(h1[j] * s2[j, k] for j in range(10)) + bs2[k], 0.0)
          for k in range(10)]
    h3 = [jnp.maximum(sum(h2[j] * s3[j, k] for j in range(10)) + bs3[k], 0.0)
          for k in range(10)]
    o = sum(h3[j] * s4[j, 0] for j in range(10)) + bs4[0]
    o_ref[...] = 1.0 / (1.0 + jnp.exp(-o))


def _tail_call(x2d, t2d, S1, bs1, S2, bs2, S3, bs3, S4, bs4):
    smem = pl.BlockSpec(memory_space=pltpu.SMEM)
    return pl.pallas_call(
        _tail_body,
        out_shape=jax.ShapeDtypeStruct(x2d.shape, jnp.float32),
        in_specs=[pl.BlockSpec(memory_space=pltpu.VMEM),
                  pl.BlockSpec(memory_space=pltpu.VMEM),
                  smem, smem, smem, smem, smem, smem, smem, smem],
        out_specs=pl.BlockSpec(memory_space=pltpu.VMEM),
    )(x2d, t2d, S1, bs1, S2, bs2, S3, bs3, S4, bs4)


def kernel(pixel, frame, target, U, V, Up1, Up2, Vp1, Vp2, W1, b1, W2, b2,
           S1, bs1, S2, bs2, S3, bs3, S4, bs4):
    pixel_i = pixel.astype(jnp.int32)
    frame_i = frame.astype(jnp.int32)
    w1 = W1.reshape(-1)
    wa = w1[:D].reshape(D, 1)
    wb = w1[D:2 * D].reshape(D, 1)
    wc = w1[2 * D:].reshape(D, 1)
    upk, tu = _prep_px(U.T, Up1.T, Up2.T, wa, wc)
    vpk, tv = _prep_fr(V.T, Vp1.T, Vp2.T, wb)
    tv = tv + b1[0]
    wsm = jnp.broadcast_to(
        jnp.concatenate([W2.reshape(-1), b2.reshape(-1),
                         jnp.zeros((6,), jnp.float32)])[:, None], (8, L))
    x = _sc_call(pixel_i, frame_i, wsm, upk, vpk, tu, tv)
    x2d = x.reshape(128, 128)
    t2d = target.reshape(128, 128)
    s2d = _tail_call(x2d, t2d, S1, bs1, S2, bs2, S3, bs3, S4, bs4)
    return (x.reshape(B, 1), s2d.reshape(B, 1))


# speedup vs baseline: 1.2710x; 1.0385x over previous
import functools

import jax
import jax.numpy as jnp
from jax import lax
from jax.experimental import pallas as pl
from jax.experimental.pallas import tpu as pltpu
from jax.experimental.pallas import tpu_sc as plsc

B = 16384
D = 64
NP = 65536
NF = 10000
NC = 2
NS = 16
L = 16
NW = NC * NS
BPW = B // NW
CH = 64
NCH = BPW // CH
PBLK = 8192
FBLK = 1000


def _prep_px_body(ut_ref, p1t_ref, p2t_ref, wa_ref, wc_ref, pk_ref, t_ref):
    wc = wc_ref[...]
    a1 = jnp.maximum(p1t_ref[...], 0.0) * wc
    a2 = jnp.maximum(p2t_ref[...], 0.0) * wc
    eye = (lax.broadcasted_iota(jnp.int32, (D, D), 0)
           == lax.broadcasted_iota(jnp.int32, (D, D), 1)).astype(jnp.float32)
    dn = (((0,), (0,)), ((), ()))
    a1t = lax.dot_general(a1, eye, dn, precision=None)
    a2t = lax.dot_general(a2, eye, dn, precision=None)
    pk_ref[...] = jnp.concatenate([a1t, a2t], axis=1)
    t_ref[...] = jnp.sum(ut_ref[...] * wa_ref[...], axis=0)[None, None, :]


def _prep_px(ut, p1t, p2t, wa, wc):
    g = NP // PBLK
    spec = pl.BlockSpec((D, PBLK), lambda i: (0, i))
    wspec = pl.BlockSpec((D, 1), lambda i: (0, 0))
    pk, t = pl.pallas_call(
        _prep_px_body,
        grid=(g,),
        in_specs=[spec, spec, spec, wspec, wspec],
        out_specs=[pl.BlockSpec((PBLK, 2 * D), lambda i: (i, 0)),
                   pl.BlockSpec((1, 1, PBLK), lambda i: (i, 0, 0))],
        out_shape=[jax.ShapeDtypeStruct((NP, 2 * D), jnp.float32),
                   jax.ShapeDtypeStruct((g, 1, PBLK), jnp.float32)],
    )(ut, p1t, p2t, wa, wc)
    return pk, t.reshape(NP)


def _prep_fr_body(vt_ref, p1t_ref, p2t_ref, wb_ref, pk_ref, t_ref):
    a1 = jnp.maximum(p1t_ref[...], 0.0)
    a2 = jnp.maximum(p2t_ref[...], 0.0)
    eye = (lax.broadcasted_iota(jnp.int32, (D, D), 0)
           == lax.broadcasted_iota(jnp.int32, (D, D), 1)).astype(jnp.float32)
    dn = (((0,), (0,)), ((), ()))
    a1t = lax.dot_general(a1, eye, dn, precision=None)
    a2t = lax.dot_general(a2, eye, dn, precision=None)
    pk_ref[...] = jnp.concatenate([a1t, a2t], axis=1)
    t_ref[...] = jnp.sum(vt_ref[...] * wb_ref[...], axis=0)[None, None, :]


def _prep_fr(vt, p1t, p2t, wb):
    spec = pl.BlockSpec((D, NF), lambda: (0, 0))
    wspec = pl.BlockSpec((D, 1), lambda: (0, 0))
    pk, t = pl.pallas_call(
        _prep_fr_body,
        in_specs=[spec, spec, spec, wspec],
        out_specs=[pl.BlockSpec((NF, 2 * D), lambda: (0, 0)),
                   pl.BlockSpec((1, 1, NF), lambda: (0, 0, 0))],
        out_shape=[jax.ShapeDtypeStruct((NF, 2 * D), jnp.float32),
                   jax.ShapeDtypeStruct((1, 1, NF), jnp.float32)],
    )(vt, p1t, p2t, wb)
    return pk, t.reshape(NF)


def _sc_body(pixel_h, frame_h, wv_h, upk_h, vpk_h, tu_h, tv_h,
             x_h,
             idxp, idxf, bu, bv, btu, btv, wv, zbuf, xbuf, sems):
    wid = lax.axis_index("s") * NC + lax.axis_index("c")
    base = wid * BPW
    pltpu.sync_copy(wv_h, wv)
    iota = lax.iota(jnp.int32, L)
    m15 = iota == jnp.full((L,), L - 1, jnp.int32)
    zero = jnp.zeros((L,), jnp.float32)
    one = jnp.full((L,), 1.0, jnp.float32)
    w2v = wv[0]
    b2v = wv[1]

    pltpu.sync_copy(pixel_h.at[pl.ds(base, BPW)], idxp)
    pltpu.sync_copy(frame_h.at[pl.ds(base, BPW)], idxf)

    def fire(c, s):
        ip = idxp.at[pl.ds(c * CH, CH)]
        if_ = idxf.at[pl.ds(c * CH, CH)]
        pltpu.async_copy(upk_h.at[ip], bu.at[s], sems.at[s])
        pltpu.async_copy(vpk_h.at[if_], bv.at[s], sems.at[s])
        pltpu.async_copy(tu_h.at[ip], btu.at[s], sems.at[s])
        pltpu.async_copy(tv_h.at[if_], btv.at[s], sems.at[s])

    def drain(s):
        ip = idxp.at[pl.ds(0, CH)]
        if_ = idxf.at[pl.ds(0, CH)]
        pltpu.make_async_copy(upk_h.at[ip], bu.at[s], sems.at[s]).wait()
        pltpu.make_async_copy(vpk_h.at[if_], bv.at[s], sems.at[s]).wait()
        pltpu.make_async_copy(tu_h.at[ip], btu.at[s], sems.at[s]).wait()
        pltpu.make_async_copy(tv_h.at[if_], btv.at[s], sems.at[s]).wait()

    fire(0, 0)
    for c in range(NCH):
        s = c % 2
        drain(s)
        if c + 1 < NCH:
            fire(c + 1, 1 - s)
        bus = bu.at[s]
        bvs = bv.at[s]

        @plsc.parallel_loop(0, CH, step=1)
        def _elem(e):
            p = bus[e, pl.ds(0, L)] * bvs[e, pl.ds(0, L)]
            for k in range(1, 2 * D // L):
                p = p + bus[e, pl.ds(k * L, L)] * bvs[e, pl.ds(k * L, L)]
            cs = plsc.cumsum(p)
            idx = jnp.full((L,), e, jnp.int32)
            plsc.store_scatter(zbuf, [idx], cs, mask=m15)

        @plsc.parallel_loop(0, CH, step=L)
        def _tailg(gbase):
            gl = pl.ds(gbase, L)
            z = zbuf[gl] + btu[s, gl] + btv[s, gl]
            h = jnp.maximum(z, zero)
            t = h * w2v + b2v
            xbuf[pl.ds(c * CH + gbase, L)] = one / (one + jnp.exp(-t))

    pltpu.sync_copy(xbuf, x_h.at[pl.ds(base, BPW)])


_sc_call = functools.partial(
    pl.kernel,
    out_type=jax.ShapeDtypeStruct((B,), jnp.float32),
    mesh=plsc.VectorSubcoreMesh(
        core_axis_name="c", subcore_axis_name="s", num_cores=NC,
        num_subcores=NS),
    scratch_types=[
        pltpu.VMEM((BPW,), jnp.int32),
        pltpu.VMEM((BPW,), jnp.int32),
        pltpu.VMEM((2, CH, 2 * D), jnp.float32),
        pltpu.VMEM((2, CH, 2 * D), jnp.float32),
        pltpu.VMEM((2, CH), jnp.float32),
        pltpu.VMEM((2, CH), jnp.float32),
        pltpu.VMEM((8, L), jnp.float32),
        pltpu.VMEM((CH,), jnp.float32),
        pltpu.VMEM((BPW,), jnp.float32),
        pltpu.SemaphoreType.DMA((2,)),
    ],
    compiler_params=pltpu.CompilerParams(
        needs_layout_passes=False, use_tc_tiling_on_sc=False),
)(_sc_body)


def _tail_body(x_ref, t_ref, s1, bs1, s2, bs2, s3, bs3, s4, bs4, o_ref):
    s = t_ref[...] - x_ref[...]
    h1 = [jnp.maximum(s * s1[0, k] + bs1[k], 0.0) for k in range(10)]
    h2 = [jnp.maximum(sum(h1[j] * s2[j, k] for j in range(10)) + bs2[k], 0.0)
          for k in range(10)]
    h3 = [jnp.maximum(sum(h2[j] * s3[j, k] for j in range(10)) + bs3[k], 0.0)
          for k in range(10)]
    o = sum(h3[j] * s4[j, 0] for j in range(10)) + bs4[0]
    o_ref[...] = 1.0 / (1.0 + jnp.exp(-o))


def _tail_call(x2d, t2d, S1, bs1, S2, bs2, S3, bs3, S4, bs4):
    smem = pl.BlockSpec(memory_space=pltpu.SMEM)
    return pl.pallas_call(
        _tail_body,
        out_shape=jax.ShapeDtypeStruct(x2d.shape, jnp.float32),
        in_specs=[pl.BlockSpec(memory_space=pltpu.VMEM),
                  pl.BlockSpec(memory_space=pltpu.VMEM),
                  smem, smem, smem, smem, smem, smem, smem, smem],
        out_specs=pl.BlockSpec(memory_space=pltpu.VMEM),
    )(x2d, t2d, S1, bs1, S2, bs2, S3, bs3, S4, bs4)


def kernel(pixel, frame, target, U, V, Up1, Up2, Vp1, Vp2, W1, b1, W2, b2,
           S1, bs1, S2, bs2, S3, bs3, S4, bs4):
    pixel_i = pixel.astype(jnp.int32)
    frame_i = frame.astype(jnp.int32)
    w1 = W1.reshape(-1)
    wa = w1[:D].reshape(D, 1)
    wb = w1[D:2 * D].reshape(D, 1)
    wc = w1[2 * D:].reshape(D, 1)
    upk, tu = _prep_px(U.T, Up1.T, Up2.T, wa, wc)
    vpk, tv = _prep_fr(V.T, Vp1.T, Vp2.T, wb)
    tv = tv + b1[0]
    wsm = jnp.broadcast_to(
        jnp.concatenate([W2.reshape(-1), b2.reshape(-1),
                         jnp.zeros((6,), jnp.float32)])[:, None], (8, L))
    x = _sc_call(pixel_i, frame_i, wsm, upk, vpk, tu, tv)
    x2d = x.reshape(128, 128)
    t2d = target.reshape(128, 128)
    s2d = _tail_call(x2d, t2d, S1, bs1, S2, bs2, S3, bs3, S4, bs4)
    return (x.reshape(B, 1), s2d.reshape(B, 1))
